# baseline (speedup 1.0000x reference)
import jax
import jax.numpy as jnp
from jax import lax
from jax.experimental import pallas as pl
from jax.experimental.pallas import tpu as pltpu
from jax.experimental.pallas import tpu_sc as plsc

NC = 2
NS = 16
NW = NC * NS
B = 16384
D = 32
USER_ROWS = 1000000
BPW = B // NW
CHUNK = 128
NCHUNK = BPW // CHUNK


def _gmf_body(user_hbm, item_hbm, uwt_hbm, iwt_hbm, params_hbm, out_hbm,
              uidx_v, iidx_v, ubuf_v, ibuf_v, params_v, out_v, sem):
    wid = lax.axis_index("s") * NC + lax.axis_index("c")
    pltpu.sync_copy(user_hbm.at[wid], uidx_v)
    pltpu.sync_copy(item_hbm.at[wid], iidx_v)
    pltpu.sync_copy(params_hbm, params_v)

    copies = []
    for d in range(D):
        for j in range(NCHUNK):
            copies.append(pltpu.async_copy(
                uwt_hbm.at[d].at[uidx_v.at[j]],
                ubuf_v.at[d, pl.ds(j * CHUNK, CHUNK)], sem))
            copies.append(pltpu.async_copy(
                iwt_hbm.at[d].at[iidx_v.at[j]],
                ibuf_v.at[d, pl.ds(j * CHUNK, CHUNK)], sem))
    for c in copies:
        c.wait()

    w_lo = params_v[pl.ds(0, 16)]
    w_hi = params_v[pl.ds(16, 16)]
    bias = params_v[pl.ds(32, 16)][0]
    wd = [w_lo[d] for d in range(16)] + [w_hi[d - 16] for d in range(16, D)]

    def body(g, carry):
        off = g * 16
        acc = jnp.zeros((16,), jnp.float32)
        for d in range(D):
            u = ubuf_v[d, pl.ds(off, 16)]
            i = ibuf_v[d, pl.ds(off, 16)]
            acc = acc + (u * i) * wd[d]
        out_v[pl.ds(off, 16)] = acc + bias
        return carry

    lax.fori_loop(0, BPW // 16, body, 0)
    pltpu.sync_copy(out_v, out_hbm.at[wid])


def kernel(user, item, embed_user_weight, embed_item_weight, predict_weight,
           predict_bias):
    uw_c, iw_c = jax.lax.optimization_barrier(
        (embed_user_weight.T.reshape(250000, 128),
         embed_item_weight.T.reshape(250000, 128)))
    uw_t = uw_c.reshape(D, USER_ROWS)
    iw_t = iw_c.reshape(D, USER_ROWS)
    user3 = user.reshape(NW, NCHUNK, CHUNK)
    item3 = item.reshape(NW, NCHUNK, CHUNK)
    params = jnp.concatenate([
        predict_weight.reshape(D), predict_bias,
        jnp.zeros((15,), jnp.float32)])
    mesh = plsc.VectorSubcoreMesh(core_axis_name="c", subcore_axis_name="s")
    k = pl.kernel(
        _gmf_body,
        out_type=jax.ShapeDtypeStruct((NW, BPW), jnp.float32),
        mesh=mesh,
        scratch_types=[
            pltpu.VMEM((NCHUNK, CHUNK), jnp.int32),
            pltpu.VMEM((NCHUNK, CHUNK), jnp.int32),
            pltpu.VMEM((D, BPW), jnp.float32),
            pltpu.VMEM((D, BPW), jnp.float32),
            pltpu.VMEM((48,), jnp.float32),
            pltpu.VMEM((BPW,), jnp.float32),
            pltpu.SemaphoreType.DMA,
        ],
        compiler_params=pltpu.CompilerParams(
            needs_layout_passes=False, use_tc_tiling_on_sc=False),
    )
    out = k(user3, item3, uw_t, iw_t, params)
    return out.reshape(B)


# speedup vs baseline: 7.7038x; 7.7038x over previous
import jax
import jax.numpy as jnp
from jax import lax
from jax.experimental import pallas as pl
from jax.experimental.pallas import tpu as pltpu
from jax.experimental.pallas import tpu_sc as plsc

NC = 2
NS = 16
NW = NC * NS
B = 16384
D = 32
PACK = 128 // D
BPW = B // NW
CHUNK = 128
NCHUNK = BPW // CHUNK


def _gmf_body(user_hbm, item_hbm, uw_hbm, iw_hbm, params_hbm, out_hbm,
              uidx_v, iidx_v, udma_v, idma_v, ucol_v, icol_v,
              ub0, ub1, ib0, ib1, params_v, out_v, sem0, sem1):
    wid = lax.axis_index("s") * NC + lax.axis_index("c")
    pltpu.sync_copy(user_hbm.at[wid], uidx_v)
    pltpu.sync_copy(item_hbm.at[wid], iidx_v)
    pltpu.sync_copy(params_hbm, params_v)

    for j in range(NCHUNK):
        for k in range(CHUNK // 16):
            uv = uidx_v[j, pl.ds(k * 16, 16)]
            iv = iidx_v[j, pl.ds(k * 16, 16)]
            udma_v[j, pl.ds(k * 16, 16)] = uv >> 2
            idma_v[j, pl.ds(k * 16, 16)] = iv >> 2
            ucol_v[pl.ds(j * CHUNK + k * 16, 16)] = (uv & 3) * D
            icol_v[pl.ds(j * CHUNK + k * 16, 16)] = (iv & 3) * D

    w_lo = params_v[pl.ds(0, 16)]
    w_hi = params_v[pl.ds(16, 16)]
    bias = params_v[pl.ds(32, 16)][0]
    wd = [w_lo[d] for d in range(16)] + [w_hi[d - 16] for d in range(16, D)]
    lane = jnp.arange(16, dtype=jnp.int32)

    ubufs = [ub0, ub1]
    ibufs = [ib0, ib1]
    sems = [sem0, sem1]

    def fire(c):
        s = sems[c % 2]
        return (pltpu.async_copy(uw_hbm.at[udma_v.at[c]], ubufs[c % 2], s),
                pltpu.async_copy(iw_hbm.at[idma_v.at[c]], ibufs[c % 2], s))

    pending = fire(0)
    for c in range(NCHUNK):
        nxt = fire(c + 1) if c + 1 < NCHUNK else None
        for p in pending:
            p.wait()
        ubuf = ubufs[c % 2]
        ibuf = ibufs[c % 2]

        def body(g, carry):
            rows = g * 16 + lane
            ucol = ucol_v[pl.ds(c * CHUNK + g * 16, 16)]
            icol = icol_v[pl.ds(c * CHUNK + g * 16, 16)]
            acc = jnp.zeros((16,), jnp.float32)
            for d in range(D):
                u = plsc.load_gather(ubuf, [rows, ucol + d])
                i = plsc.load_gather(ibuf, [rows, icol + d])
                acc = acc + (u * i) * wd[d]
            out_v[pl.ds(c * CHUNK + g * 16, 16)] = acc + bias
            return carry

        lax.fori_loop(0, CHUNK // 16, body, 0)
        pending = nxt

    pltpu.sync_copy(out_v, out_hbm.at[wid])


def kernel(user, item, embed_user_weight, embed_item_weight, predict_weight,
           predict_bias):
    uw_p = embed_user_weight.reshape(250000, 128)
    iw_p = embed_item_weight.reshape(250000, 128)
    user3 = user.reshape(NW, NCHUNK, CHUNK)
    item3 = item.reshape(NW, NCHUNK, CHUNK)
    params = jnp.concatenate([
        predict_weight.reshape(D), predict_bias,
        jnp.zeros((15,), jnp.float32)])
    mesh = plsc.VectorSubcoreMesh(core_axis_name="c", subcore_axis_name="s")
    k = pl.kernel(
        _gmf_body,
        out_type=jax.ShapeDtypeStruct((NW, BPW), jnp.float32),
        mesh=mesh,
        scratch_types=[
            pltpu.VMEM((NCHUNK, CHUNK), jnp.int32),
            pltpu.VMEM((NCHUNK, CHUNK), jnp.int32),
            pltpu.VMEM((NCHUNK, CHUNK), jnp.int32),
            pltpu.VMEM((NCHUNK, CHUNK), jnp.int32),
            pltpu.VMEM((BPW,), jnp.int32),
            pltpu.VMEM((BPW,), jnp.int32),
            pltpu.VMEM((CHUNK, 128), jnp.float32),
            pltpu.VMEM((CHUNK, 128), jnp.float32),
            pltpu.VMEM((CHUNK, 128), jnp.float32),
            pltpu.VMEM((CHUNK, 128), jnp.float32),
            pltpu.VMEM((48,), jnp.float32),
            pltpu.VMEM((BPW,), jnp.float32),
            pltpu.SemaphoreType.DMA,
            pltpu.SemaphoreType.DMA,
        ],
        compiler_params=pltpu.CompilerParams(
            needs_layout_passes=False, use_tc_tiling_on_sc=True),
    )
    out = k(user3, item3, uw_p, iw_p, params)
    return out.reshape(B)
